# baseline (speedup 1.0000x reference)
import functools

import jax
import jax.numpy as jnp
from jax import lax
from jax.experimental import pallas as pl
from jax.experimental.pallas import tpu as pltpu
from jax.experimental.pallas import tpu_sc as plsc


def _sc_gather(sup, idx_flat, batch, hist, dim):
    info = plsc.get_sparse_core_info()
    nc, ns, lanes = info.num_cores, info.num_subcores, info.num_lanes
    nw = nc * ns
    bb_per_w = batch // (nw * 128)
    n_units = bb_per_w * hist
    kgrp = 128 // lanes
    dsub = dim // 8
    mesh = plsc.VectorSubcoreMesh(core_axis_name="c", subcore_axis_name="s")

    @functools.partial(
        pl.kernel,
        mesh=mesh,
        compiler_params=pltpu.CompilerParams(use_tc_tiling_on_sc=True,
                                             needs_layout_passes=False),
        out_type=jax.ShapeDtypeStruct((hist, dim, batch), jnp.float32),
        scratch_types=[
            pltpu.VMEM((128 * hist,), jnp.int32),
            pltpu.VMEM((2, 128), jnp.int32),
            pltpu.VMEM((2, 128), jnp.int32),
            pltpu.VMEM((2, 128, 128), jnp.float32),
            pltpu.VMEM((2, dim, 128), jnp.float32),
            pltpu.SemaphoreType.DMA((2,)),
            pltpu.SemaphoreType.DMA((2,)),
        ],
    )
    def k(sup_hbm, idx_hbm, out_hbm, idxblk, srbuf, offbuf, gbuf, obuf,
          gsem, osem):
        wid = lax.axis_index("s") * nc + lax.axis_index("c")
        iota = lax.iota(jnp.int32, lanes)

        def load_idxblk(bb):
            pltpu.sync_copy(idx_hbm.at[pl.ds(bb * 128 * hist, 128 * hist)],
                            idxblk)

        def build(u, slot):
            h = lax.rem(u, hist)
            for kk in range(kgrp):
                ivec = (kk * lanes + iota) * hist + h
                v = plsc.load_gather(idxblk, [ivec])
                srbuf[slot, pl.ds(kk * lanes, lanes)] = v >> 2
                offbuf[slot, pl.ds(kk * lanes, lanes)] = (v & 3) * dim

        def start_gather(slot):
            pltpu.async_copy(sup_hbm.at[srbuf.at[slot]], gbuf.at[slot],
                             gsem.at[slot])

        def wait_gather(slot):
            pltpu.make_async_copy(sup_hbm.at[pl.ds(0, 128)], gbuf.at[slot],
                                  gsem.at[slot]).wait()

        def extract(slot):
            slot_v = jnp.full((lanes,), slot, jnp.int32)
            for kk in range(kgrp):
                off = offbuf[slot, pl.ds(kk * lanes, lanes)]
                row = jnp.full((lanes,), kk * lanes, jnp.int32) + iota
                for d in range(dim):
                    val = plsc.load_gather(gbuf, [slot_v, row, off + d])
                    obuf[slot, d, pl.ds(kk * lanes, lanes)] = val

        def start_stores(u, slot):
            h = lax.rem(u, hist)
            bb = wid * bb_per_w + u // hist
            for db in range(dsub):
                pltpu.async_copy(
                    obuf.at[slot, pl.ds(db * 8, 8)],
                    out_hbm.at[h, pl.ds(db * 8, 8), pl.ds(bb * 128, 128)],
                    osem.at[slot])

        def wait_stores(slot):
            pltpu.make_async_copy(sup_hbm.at[pl.ds(0, dim // 8 * 8 * 128 // 128)],
                                  obuf.at[slot], osem.at[slot]).wait()

        load_idxblk(wid * bb_per_w)
        build(0, 0)
        start_gather(0)

        def body(u, carry):
            slot = lax.rem(u, 2)
            nslot = 1 - slot
            nxt = u + 1

            @pl.when(nxt < n_units)
            def _prep():
                @pl.when(lax.rem(nxt, hist) == 0)
                def _reload():
                    load_idxblk(wid * bb_per_w + nxt // hist)
                build(nxt, nslot)
                start_gather(nslot)

            wait_gather(slot)
            @pl.when(u >= 2)
            def _drain():
                wait_stores(slot)
            extract(slot)
            start_stores(u, slot)
            return carry

        lax.fori_loop(0, n_units, body, 0)
        wait_stores(0)
        wait_stores(1)

    return k(sup, idx_flat)


def kernel(indices, weight):
    b, h = indices.shape
    dim = weight.shape[1]
    idx_flat = indices.reshape(-1).astype(jnp.int32)
    sup = weight.reshape(-1, 128)
    out = _sc_gather(sup, idx_flat, b, h, dim)
    return out.transpose(2, 0, 1)


# speedup vs baseline: 1.1258x; 1.1258x over previous
import functools

import jax
import jax.numpy as jnp
from jax import lax
from jax.experimental import pallas as pl
from jax.experimental.pallas import tpu as pltpu
from jax.experimental.pallas import tpu_sc as plsc


def _sc_gather(table, idx_flat, batch, hist, dim):
    info = plsc.get_sparse_core_info()
    nc, ns = info.num_cores, info.num_subcores
    nw = nc * ns
    b_per_w = batch // nw
    bchunk = 32
    chunk = bchunk * hist
    n_chunks = b_per_w // bchunk
    mesh = plsc.VectorSubcoreMesh(core_axis_name="c", subcore_axis_name="s")

    @functools.partial(
        pl.kernel,
        mesh=mesh,
        compiler_params=pltpu.CompilerParams(use_tc_tiling_on_sc=False),
        out_type=jax.ShapeDtypeStruct((batch, hist, dim), jnp.float32),
        scratch_types=[
            pltpu.VMEM((2, chunk), jnp.int32),
            pltpu.VMEM((2, chunk, dim), jnp.float32),
            pltpu.SemaphoreType.DMA((2,)),
            pltpu.SemaphoreType.DMA((2,)),
            pltpu.SemaphoreType.DMA((2,)),
        ],
    )
    def k(table_hbm, idx_hbm, out_hbm, idx_v, rows_v, idx_sem, gat_sem, out_sem):
        wid = lax.axis_index("s") * nc + lax.axis_index("c")
        base = wid * b_per_w

        def start_idx_load(i, slot):
            return pltpu.async_copy(
                idx_hbm.at[pl.ds((base + i * bchunk) * hist, chunk)],
                idx_v.at[slot], idx_sem.at[slot])

        def start_gather(slot):
            pltpu.async_copy(table_hbm.at[idx_v.at[slot]], rows_v.at[slot],
                             gat_sem.at[slot])

        def start_store(i, slot):
            b0 = base + i * bchunk
            for j in range(bchunk):
                pltpu.async_copy(
                    rows_v.at[slot, pl.ds(j * hist, hist)],
                    out_hbm.at[b0 + j], out_sem.at[slot])

        def wait_gather(slot):
            pltpu.make_async_copy(
                table_hbm.at[pl.ds(0, chunk)], rows_v.at[slot],
                gat_sem.at[slot]).wait()

        def wait_store(slot):
            pltpu.make_async_copy(
                table_hbm.at[pl.ds(0, chunk)], rows_v.at[slot],
                out_sem.at[slot]).wait()

        start_idx_load(0, 0).wait()
        start_gather(0)
        start_idx_load(1, 1).wait()
        start_gather(1)

        def body(i, carry):
            slot = lax.rem(i, 2)
            wait_gather(slot)
            start_store(i, slot)
            start_idx_load(i + 2, slot).wait()
            wait_store(slot)
            start_gather(slot)
            return carry

        lax.fori_loop(0, n_chunks - 2, body, 0)

        def tail(i):
            slot = lax.rem(i, 2)
            wait_gather(slot)
            start_store(i, slot)
            wait_store(slot)

        tail(n_chunks - 2)
        tail(n_chunks - 1)

    return k(table, idx_flat)


def kernel(indices, weight):
    b, h = indices.shape
    dim = weight.shape[1]
    idx_flat = indices.reshape(-1).astype(jnp.int32)
    return _sc_gather(weight, idx_flat, b, h, dim)
